# baseline (speedup 1.0000x reference)
import functools

import jax
import jax.numpy as jnp
from jax import lax
from jax.experimental import pallas as pl
from jax.experimental.pallas import tpu as pltpu
from jax.experimental.pallas import tpu_sc as plsc

N = 50000
D = 256
G = 128

B_TC = 2048
TBLK = 14
NT = TBLK * B_TC
NS = N - NT

NCORES = 2
NTILES = 16
DC = D // NCORES
LANES = 16
NV = DC // LANES
ROWS_PER_TILE = 1336
OVERLAP = NTILES * ROWS_PER_TILE - NS
CHUNK = 128
NFULL = ROWS_PER_TILE // CHUNK
TAIL = ROWS_PER_TILE - NFULL * CHUNK
NCHUNK = NFULL + 1
KPH = 4
AROWS = G * KPH + 16


def _body(
    feats_hbm, ids_hbm, out_hbm,
    ids_v, fbuf, wbuf, zbuf, acc, sg0, sg1,
):
    cid = lax.axis_index("c")
    sid = lax.axis_index("s")
    col0 = cid * DC
    base = NT + jnp.minimum(sid * ROWS_PER_TILE, NS - ROWS_PER_TILE)
    sgs = (sg0, sg1)

    def gather(j, b, rows=CHUNK):
        return pltpu.make_async_copy(
            feats_hbm.at[pl.ds(base + j * CHUNK, rows), pl.ds(col0, DC)],
            fbuf.at[b] if rows == CHUNK else fbuf.at[b, pl.ds(0, rows)],
            sgs[b],
        )

    gather(0, 0).start()
    gather(1, 1).start()

    zero = jnp.zeros((LANES,), jnp.float32)
    for r in range(2 * KPH):
        for j in range(NV):
            wbuf[r, pl.ds(j * LANES, LANES)] = zero
    for q in range(4):
        pltpu.sync_copy(
            wbuf.at[pl.ds(0, 8)], acc.at[pl.ds(sid * 8 * KPH + q * 8, 8)]
        )

    @pl.when(sid < 2)
    def _():
        pltpu.sync_copy(
            wbuf.at[pl.ds(0, 8)], acc.at[pl.ds(G * KPH + sid * 8, 8)]
        )

    pltpu.sync_copy(ids_hbm.at[sid], ids_v)
    plsc.subcore_barrier()

    def pair_body(k, carry):
        for b in range(2):
            j = 2 * k + b
            gather(j, b).wait()
            pltpu.sync_copy(fbuf.at[b], acc.at[ids_v.at[j]], add=True)

            @pl.when(j + 2 < NFULL)
            def _():
                gather(j + 2, b).start()

        return carry

    lax.fori_loop(0, NFULL // 2, pair_body, 0)

    gather(NFULL, 0, TAIL).start()
    gather(NFULL, 0, TAIL).wait()
    pltpu.sync_copy(fbuf.at[0], acc.at[ids_v.at[NFULL]], add=True)

    plsc.subcore_barrier()

    pltpu.sync_copy(acc.at[pl.ds(sid * 8 * KPH, 8 * KPH)], wbuf)
    for r in range(8):
        for j in range(NV):
            s = wbuf[r * KPH, pl.ds(j * LANES, LANES)]
            for q in range(1, KPH):
                s = s + wbuf[r * KPH + q, pl.ds(j * LANES, LANES)]
            zbuf[r, pl.ds(j * LANES, LANES)] = s
    pltpu.sync_copy(
        zbuf,
        out_hbm.at[pl.ds(sid * 8, 8), pl.ds(col0, DC)],
    )


def _tc_body(ids_ref, feats_ref, out_ref):
    i = pl.program_id(0)
    ids = ids_ref[0, 0, :]
    onehot = (
        lax.broadcasted_iota(jnp.int32, (B_TC, G), 1) == ids[:, None]
    ).astype(jnp.float32)
    part = lax.dot_general(
        onehot,
        feats_ref[...],
        (((0,), (0,)), ((), ())),
        preferred_element_type=jnp.float32,
        precision=lax.Precision.HIGHEST,
    )

    @pl.when(i == 0)
    def _():
        out_ref[...] = part

    @pl.when(i > 0)
    def _():
        out_ref[...] = out_ref[...] + part


def kernel(feats, segment_ids, num_segments):
    ids = segment_ids.astype(jnp.int32) + (
        jnp.asarray(num_segments, jnp.int32) - G
    )
    ids_tc = ids[:NT].reshape(TBLK, 1, B_TC)
    ids_sc = ids[NT:]
    main = ids_sc[: (NTILES - 1) * ROWS_PER_TILE].reshape(
        NTILES - 1, ROWS_PER_TILE
    )
    last = ids_sc[NS - ROWS_PER_TILE :]
    last = jnp.where(
        jnp.arange(ROWS_PER_TILE, dtype=jnp.int32) < OVERLAP, G, last
    )
    ids = jnp.concatenate([main, last[None]], axis=0)
    phase = (
        jnp.arange(NTILES * ROWS_PER_TILE, dtype=jnp.int32) % KPH
    ).reshape(NTILES, ROWS_PER_TILE)
    ids = jnp.where(ids >= G, G * KPH, ids * KPH + phase)
    ids = jnp.pad(
        ids,
        ((0, 0), (0, NCHUNK * CHUNK - ROWS_PER_TILE)),
        constant_values=G * KPH,
    )
    ids = ids.reshape(NTILES, NCHUNK, CHUNK)

    mesh = plsc.VectorSubcoreMesh(core_axis_name="c", subcore_axis_name="s")
    run = functools.partial(
        pl.kernel,
        mesh=mesh,
        out_type=jax.ShapeDtypeStruct((G, D), jnp.float32),
        scratch_types=[
            pltpu.VMEM((NCHUNK, CHUNK), jnp.int32),
            pltpu.VMEM((2, CHUNK, DC), jnp.float32),
            pltpu.VMEM((8 * KPH, DC), jnp.float32),
            pltpu.VMEM((8, DC), jnp.float32),
            pltpu.VMEM_SHARED((AROWS, DC), jnp.float32),
            pltpu.SemaphoreType.DMA,
            pltpu.SemaphoreType.DMA,
        ],
    )(_body)
    tc_part = pl.pallas_call(
        _tc_body,
        grid=(TBLK,),
        in_specs=[
            pl.BlockSpec((1, 1, B_TC), lambda i: (i, 0, 0)),
            pl.BlockSpec((B_TC, D), lambda i: (i, 0)),
        ],
        out_specs=pl.BlockSpec((G, D), lambda i: (0, 0)),
        out_shape=jax.ShapeDtypeStruct((G, D), jnp.float32),
        compiler_params=pltpu.CompilerParams(
            dimension_semantics=("arbitrary",),
        ),
    )(ids_tc, feats)
    return run(feats, ids) + tc_part


# speedup vs baseline: 1.0667x; 1.0667x over previous
import functools

import jax
import jax.numpy as jnp
from jax import lax
from jax.experimental import pallas as pl
from jax.experimental.pallas import tpu as pltpu
from jax.experimental.pallas import tpu_sc as plsc

N = 50000
D = 256
G = 128

B_TC = 2048
TBLK = 16
NT = TBLK * B_TC
NS = N - NT

NCORES = 2
NTILES = 16
DC = D // NCORES
LANES = 16
NV = DC // LANES
ROWS_PER_TILE = 1080
OVERLAP = NTILES * ROWS_PER_TILE - NS
CHUNK = 128
NFULL = ROWS_PER_TILE // CHUNK
TAIL = ROWS_PER_TILE - NFULL * CHUNK
NCHUNK = NFULL + 1
KPH = 4
AROWS = G * KPH + 16


def _body(
    feats_hbm, ids_hbm, out_hbm,
    ids_v, fbuf, wbuf, zbuf, acc, sg0, sg1,
):
    cid = lax.axis_index("c")
    sid = lax.axis_index("s")
    col0 = cid * DC
    base = NT + jnp.minimum(sid * ROWS_PER_TILE, NS - ROWS_PER_TILE)
    sgs = (sg0, sg1)

    def gather(j, b, rows=CHUNK):
        return pltpu.make_async_copy(
            feats_hbm.at[pl.ds(base + j * CHUNK, rows), pl.ds(col0, DC)],
            fbuf.at[b] if rows == CHUNK else fbuf.at[b, pl.ds(0, rows)],
            sgs[b],
        )

    gather(0, 0).start()
    gather(1, 1).start()

    zero = jnp.zeros((LANES,), jnp.float32)
    for r in range(2 * KPH):
        for j in range(NV):
            wbuf[r, pl.ds(j * LANES, LANES)] = zero
    for q in range(4):
        pltpu.sync_copy(
            wbuf.at[pl.ds(0, 8)], acc.at[pl.ds(sid * 8 * KPH + q * 8, 8)]
        )

    @pl.when(sid < 2)
    def _():
        pltpu.sync_copy(
            wbuf.at[pl.ds(0, 8)], acc.at[pl.ds(G * KPH + sid * 8, 8)]
        )

    pltpu.sync_copy(ids_hbm.at[sid], ids_v)
    plsc.subcore_barrier()

    def pair_body(k, carry):
        for b in range(2):
            j = 2 * k + b
            gather(j, b).wait()
            pltpu.sync_copy(fbuf.at[b], acc.at[ids_v.at[j]], add=True)

            @pl.when(j + 2 < NFULL)
            def _():
                gather(j + 2, b).start()

        return carry

    lax.fori_loop(0, NFULL // 2, pair_body, 0)

    gather(NFULL, 0, TAIL).start()
    gather(NFULL, 0, TAIL).wait()
    pltpu.sync_copy(fbuf.at[0], acc.at[ids_v.at[NFULL]], add=True)

    plsc.subcore_barrier()

    pltpu.sync_copy(acc.at[pl.ds(sid * 8 * KPH, 8 * KPH)], wbuf)
    for r in range(8):
        for j in range(NV):
            s = wbuf[r * KPH, pl.ds(j * LANES, LANES)]
            for q in range(1, KPH):
                s = s + wbuf[r * KPH + q, pl.ds(j * LANES, LANES)]
            zbuf[r, pl.ds(j * LANES, LANES)] = s
    pltpu.sync_copy(
        zbuf,
        out_hbm.at[pl.ds(sid * 8, 8), pl.ds(col0, DC)],
    )


def _tc_body(ids_ref, feats_ref, out_ref):
    i = pl.program_id(0)
    ids = ids_ref[0, 0, :]
    onehot = (
        lax.broadcasted_iota(jnp.int32, (B_TC, G), 1) == ids[:, None]
    ).astype(jnp.float32)
    part = lax.dot_general(
        onehot,
        feats_ref[...],
        (((0,), (0,)), ((), ())),
        preferred_element_type=jnp.float32,
    )

    @pl.when(i == 0)
    def _():
        out_ref[...] = part

    @pl.when(i > 0)
    def _():
        out_ref[...] = out_ref[...] + part


def kernel(feats, segment_ids, num_segments):
    ids = segment_ids.astype(jnp.int32) + (
        jnp.asarray(num_segments, jnp.int32) - G
    )
    ids_tc = ids[:NT].reshape(TBLK, 1, B_TC)
    ids_sc = ids[NT:]
    main = ids_sc[: (NTILES - 1) * ROWS_PER_TILE].reshape(
        NTILES - 1, ROWS_PER_TILE
    )
    last = ids_sc[NS - ROWS_PER_TILE :]
    last = jnp.where(
        jnp.arange(ROWS_PER_TILE, dtype=jnp.int32) < OVERLAP, G, last
    )
    ids = jnp.concatenate([main, last[None]], axis=0)
    phase = (
        jnp.arange(NTILES * ROWS_PER_TILE, dtype=jnp.int32) % KPH
    ).reshape(NTILES, ROWS_PER_TILE)
    ids = jnp.where(ids >= G, G * KPH, ids * KPH + phase)
    ids = jnp.pad(
        ids,
        ((0, 0), (0, NCHUNK * CHUNK - ROWS_PER_TILE)),
        constant_values=G * KPH,
    )
    ids = ids.reshape(NTILES, NCHUNK, CHUNK)

    mesh = plsc.VectorSubcoreMesh(core_axis_name="c", subcore_axis_name="s")
    run = functools.partial(
        pl.kernel,
        mesh=mesh,
        out_type=jax.ShapeDtypeStruct((G, D), jnp.float32),
        scratch_types=[
            pltpu.VMEM((NCHUNK, CHUNK), jnp.int32),
            pltpu.VMEM((2, CHUNK, DC), jnp.float32),
            pltpu.VMEM((8 * KPH, DC), jnp.float32),
            pltpu.VMEM((8, DC), jnp.float32),
            pltpu.VMEM_SHARED((AROWS, DC), jnp.float32),
            pltpu.SemaphoreType.DMA,
            pltpu.SemaphoreType.DMA,
        ],
    )(_body)
    tc_part = pl.pallas_call(
        _tc_body,
        grid=(TBLK,),
        in_specs=[
            pl.BlockSpec((1, 1, B_TC), lambda i: (i, 0, 0)),
            pl.BlockSpec((B_TC, D), lambda i: (i, 0)),
        ],
        out_specs=pl.BlockSpec((G, D), lambda i: (0, 0)),
        out_shape=jax.ShapeDtypeStruct((G, D), jnp.float32),
        compiler_params=pltpu.CompilerParams(
            dimension_semantics=("arbitrary",),
        ),
    )(ids_tc, feats)
    return run(feats, ids) + tc_part


# speedup vs baseline: 1.1042x; 1.0352x over previous
import functools

import jax
import jax.numpy as jnp
from jax import lax
from jax.experimental import pallas as pl
from jax.experimental.pallas import tpu as pltpu
from jax.experimental.pallas import tpu_sc as plsc

N = 50000
D = 256
G = 128

B_TC = 4096
TBLK = 7
NT = TBLK * B_TC
NS = N - NT

NCORES = 2
NTILES = 16
DC = D // NCORES
LANES = 16
NV = DC // LANES
ROWS_PER_TILE = 1336
OVERLAP = NTILES * ROWS_PER_TILE - NS
CHUNK = 128
NFULL = ROWS_PER_TILE // CHUNK
TAIL = ROWS_PER_TILE - NFULL * CHUNK
NCHUNK = NFULL + 1
KPH = 4
AROWS = G * KPH + 16


def _body(
    feats_hbm, ids_hbm, out_hbm,
    ids_v, fbuf, wbuf, zbuf, acc, sg0, sg1,
):
    cid = lax.axis_index("c")
    sid = lax.axis_index("s")
    col0 = cid * DC
    base = NT + jnp.minimum(sid * ROWS_PER_TILE, NS - ROWS_PER_TILE)
    sgs = (sg0, sg1)

    def gather(j, b, rows=CHUNK):
        return pltpu.make_async_copy(
            feats_hbm.at[pl.ds(base + j * CHUNK, rows), pl.ds(col0, DC)],
            fbuf.at[b] if rows == CHUNK else fbuf.at[b, pl.ds(0, rows)],
            sgs[b],
        )

    gather(0, 0).start()
    gather(1, 1).start()

    zero = jnp.zeros((LANES,), jnp.float32)
    for r in range(2 * KPH):
        for j in range(NV):
            wbuf[r, pl.ds(j * LANES, LANES)] = zero
    for q in range(4):
        pltpu.sync_copy(
            wbuf.at[pl.ds(0, 8)], acc.at[pl.ds(sid * 8 * KPH + q * 8, 8)]
        )

    @pl.when(sid < 2)
    def _():
        pltpu.sync_copy(
            wbuf.at[pl.ds(0, 8)], acc.at[pl.ds(G * KPH + sid * 8, 8)]
        )

    pltpu.sync_copy(ids_hbm.at[sid], ids_v)
    plsc.subcore_barrier()

    def pair_body(k, carry):
        for b in range(2):
            j = 2 * k + b
            gather(j, b).wait()
            pltpu.sync_copy(fbuf.at[b], acc.at[ids_v.at[j]], add=True)

            @pl.when(j + 2 < NFULL)
            def _():
                gather(j + 2, b).start()

        return carry

    lax.fori_loop(0, NFULL // 2, pair_body, 0)

    gather(NFULL, 0, TAIL).start()
    gather(NFULL, 0, TAIL).wait()
    pltpu.sync_copy(fbuf.at[0], acc.at[ids_v.at[NFULL]], add=True)

    plsc.subcore_barrier()

    pltpu.sync_copy(acc.at[pl.ds(sid * 8 * KPH, 8 * KPH)], wbuf)
    for r in range(8):
        for j in range(NV):
            s = wbuf[r * KPH, pl.ds(j * LANES, LANES)]
            for q in range(1, KPH):
                s = s + wbuf[r * KPH + q, pl.ds(j * LANES, LANES)]
            zbuf[r, pl.ds(j * LANES, LANES)] = s
    pltpu.sync_copy(
        zbuf,
        out_hbm.at[pl.ds(sid * 8, 8), pl.ds(col0, DC)],
    )


def _tc_body(ids_ref, feats_ref, out_ref):
    i = pl.program_id(0)
    ids = ids_ref[0, 0, :]
    onehot = (
        lax.broadcasted_iota(jnp.int32, (B_TC, G), 1) == ids[:, None]
    ).astype(jnp.float32)
    part = lax.dot_general(
        onehot,
        feats_ref[...],
        (((0,), (0,)), ((), ())),
        preferred_element_type=jnp.float32,
    )

    @pl.when(i == 0)
    def _():
        out_ref[...] = part

    @pl.when(i > 0)
    def _():
        out_ref[...] = out_ref[...] + part


def kernel(feats, segment_ids, num_segments):
    ids = segment_ids.astype(jnp.int32) + (
        jnp.asarray(num_segments, jnp.int32) - G
    )
    ids_tc = ids[:NT].reshape(TBLK, 1, B_TC)
    ids_sc = ids[NT:]
    main = ids_sc[: (NTILES - 1) * ROWS_PER_TILE].reshape(
        NTILES - 1, ROWS_PER_TILE
    )
    last = ids_sc[NS - ROWS_PER_TILE :]
    last = jnp.where(
        jnp.arange(ROWS_PER_TILE, dtype=jnp.int32) < OVERLAP, G, last
    )
    ids = jnp.concatenate([main, last[None]], axis=0)
    phase = (
        jnp.arange(NTILES * ROWS_PER_TILE, dtype=jnp.int32) % KPH
    ).reshape(NTILES, ROWS_PER_TILE)
    ids = jnp.where(ids >= G, G * KPH, ids * KPH + phase)
    ids = jnp.pad(
        ids,
        ((0, 0), (0, NCHUNK * CHUNK - ROWS_PER_TILE)),
        constant_values=G * KPH,
    )
    ids = ids.reshape(NTILES, NCHUNK, CHUNK)

    mesh = plsc.VectorSubcoreMesh(core_axis_name="c", subcore_axis_name="s")
    run = functools.partial(
        pl.kernel,
        mesh=mesh,
        out_type=jax.ShapeDtypeStruct((G, D), jnp.float32),
        scratch_types=[
            pltpu.VMEM((NCHUNK, CHUNK), jnp.int32),
            pltpu.VMEM((2, CHUNK, DC), jnp.float32),
            pltpu.VMEM((8 * KPH, DC), jnp.float32),
            pltpu.VMEM((8, DC), jnp.float32),
            pltpu.VMEM_SHARED((AROWS, DC), jnp.float32),
            pltpu.SemaphoreType.DMA,
            pltpu.SemaphoreType.DMA,
        ],
    )(_body)
    tc_part = pl.pallas_call(
        _tc_body,
        grid=(TBLK,),
        in_specs=[
            pl.BlockSpec((1, 1, B_TC), lambda i: (i, 0, 0)),
            pl.BlockSpec((B_TC, D), lambda i: (i, 0)),
        ],
        out_specs=pl.BlockSpec((G, D), lambda i: (0, 0)),
        out_shape=jax.ShapeDtypeStruct((G, D), jnp.float32),
        compiler_params=pltpu.CompilerParams(
            dimension_semantics=("arbitrary",),
        ),
    )(ids_tc, feats)
    return run(feats, ids) + tc_part


# speedup vs baseline: 1.1140x; 1.0089x over previous
import functools

import jax
import jax.numpy as jnp
from jax import lax
from jax.experimental import pallas as pl
from jax.experimental.pallas import tpu as pltpu
from jax.experimental.pallas import tpu_sc as plsc

N = 50000
D = 256
G = 128

B_TC = 2048
TBLK = 14
NT = TBLK * B_TC
NS = N - NT

NCORES = 2
NTILES = 16
DC = D // NCORES
LANES = 16
NV = DC // LANES
ROWS_PER_TILE = 1336
OVERLAP = NTILES * ROWS_PER_TILE - NS
CHUNK = 128
NFULL = ROWS_PER_TILE // CHUNK
TAIL = ROWS_PER_TILE - NFULL * CHUNK
NCHUNK = NFULL + 1
KPH = 4
AROWS = G * KPH + 16


def _body(
    feats_hbm, ids_hbm, out_hbm,
    ids_v, fbuf, wbuf, zbuf, acc, sg0, sg1,
):
    cid = lax.axis_index("c")
    sid = lax.axis_index("s")
    col0 = cid * DC
    base = NT + jnp.minimum(sid * ROWS_PER_TILE, NS - ROWS_PER_TILE)
    sgs = (sg0, sg1)

    def gather(j, b, rows=CHUNK):
        return pltpu.make_async_copy(
            feats_hbm.at[pl.ds(base + j * CHUNK, rows), pl.ds(col0, DC)],
            fbuf.at[b] if rows == CHUNK else fbuf.at[b, pl.ds(0, rows)],
            sgs[b],
        )

    gather(0, 0).start()
    gather(1, 1).start()

    zero = jnp.zeros((LANES,), jnp.float32)
    for r in range(2 * KPH):
        for j in range(NV):
            wbuf[r, pl.ds(j * LANES, LANES)] = zero
    for q in range(4):
        pltpu.sync_copy(
            wbuf.at[pl.ds(0, 8)], acc.at[pl.ds(sid * 8 * KPH + q * 8, 8)]
        )

    @pl.when(sid < 2)
    def _():
        pltpu.sync_copy(
            wbuf.at[pl.ds(0, 8)], acc.at[pl.ds(G * KPH + sid * 8, 8)]
        )

    pltpu.sync_copy(ids_hbm.at[sid], ids_v)
    plsc.subcore_barrier()

    def pair_body(k, carry):
        for b in range(2):
            j = 2 * k + b
            gather(j, b).wait()
            pltpu.sync_copy(fbuf.at[b], acc.at[ids_v.at[j]], add=True)

            @pl.when(j + 2 < NFULL)
            def _():
                gather(j + 2, b).start()

        return carry

    lax.fori_loop(0, NFULL // 2, pair_body, 0)

    gather(NFULL, 0, TAIL).start()
    gather(NFULL, 0, TAIL).wait()
    pltpu.sync_copy(fbuf.at[0], acc.at[ids_v.at[NFULL]], add=True)

    plsc.subcore_barrier()

    pltpu.sync_copy(acc.at[pl.ds(sid * 8 * KPH, 8 * KPH)], wbuf)
    for r in range(8):
        for j in range(NV):
            s = wbuf[r * KPH, pl.ds(j * LANES, LANES)]
            for q in range(1, KPH):
                s = s + wbuf[r * KPH + q, pl.ds(j * LANES, LANES)]
            zbuf[r, pl.ds(j * LANES, LANES)] = s
    pltpu.sync_copy(
        zbuf,
        out_hbm.at[pl.ds(sid * 8, 8), pl.ds(col0, DC)],
    )


def _tc_body(ids_ref, feats_ref, out_ref):
    i = pl.program_id(0)
    ids = ids_ref[0, 0, :]
    onehot = (
        lax.broadcasted_iota(jnp.int32, (B_TC, G), 1) == ids[:, None]
    ).astype(jnp.float32)
    part = lax.dot_general(
        onehot,
        feats_ref[...],
        (((0,), (0,)), ((), ())),
        preferred_element_type=jnp.float32,
    )

    @pl.when(i == 0)
    def _():
        out_ref[...] = part

    @pl.when(i > 0)
    def _():
        out_ref[...] = out_ref[...] + part


def kernel(feats, segment_ids, num_segments):
    ids = segment_ids.astype(jnp.int32) + (
        jnp.asarray(num_segments, jnp.int32) - G
    )
    ids_tc = ids[:NT].reshape(TBLK, 1, B_TC)
    ids_sc = ids[NT:]
    main = ids_sc[: (NTILES - 1) * ROWS_PER_TILE].reshape(
        NTILES - 1, ROWS_PER_TILE
    )
    last = ids_sc[NS - ROWS_PER_TILE :]
    last = jnp.where(
        jnp.arange(ROWS_PER_TILE, dtype=jnp.int32) < OVERLAP, G, last
    )
    ids = jnp.concatenate([main, last[None]], axis=0)
    phase = (
        jnp.arange(NTILES * ROWS_PER_TILE, dtype=jnp.int32) % KPH
    ).reshape(NTILES, ROWS_PER_TILE)
    ids = jnp.where(ids >= G, G * KPH, ids * KPH + phase)
    ids = jnp.pad(
        ids,
        ((0, 0), (0, NCHUNK * CHUNK - ROWS_PER_TILE)),
        constant_values=G * KPH,
    )
    ids = ids.reshape(NTILES, NCHUNK, CHUNK)

    mesh = plsc.VectorSubcoreMesh(core_axis_name="c", subcore_axis_name="s")
    run = functools.partial(
        pl.kernel,
        mesh=mesh,
        out_type=jax.ShapeDtypeStruct((G, D), jnp.float32),
        scratch_types=[
            pltpu.VMEM((NCHUNK, CHUNK), jnp.int32),
            pltpu.VMEM((2, CHUNK, DC), jnp.float32),
            pltpu.VMEM((8 * KPH, DC), jnp.float32),
            pltpu.VMEM((8, DC), jnp.float32),
            pltpu.VMEM_SHARED((AROWS, DC), jnp.float32),
            pltpu.SemaphoreType.DMA,
            pltpu.SemaphoreType.DMA,
        ],
    )(_body)
    tc_part = pl.pallas_call(
        _tc_body,
        grid=(TBLK,),
        in_specs=[
            pl.BlockSpec((1, 1, B_TC), lambda i: (i, 0, 0)),
            pl.BlockSpec((B_TC, D), lambda i: (i, 0)),
        ],
        out_specs=pl.BlockSpec((G, D), lambda i: (0, 0)),
        out_shape=jax.ShapeDtypeStruct((G, D), jnp.float32),
        compiler_params=pltpu.CompilerParams(
            dimension_semantics=("arbitrary",),
        ),
    )(ids_tc, feats)
    return run(feats, ids) + tc_part
